# baseline (speedup 1.0000x reference)
import functools

import jax
import jax.numpy as jnp
from jax import lax
from jax.experimental import pallas as pl
from jax.experimental.pallas import tpu as pltpu
from jax.experimental.pallas import tpu_sc as plsc

N_OUT = 400000
D = 128
NC = 2
NS = 16
NW = NC * NS
BLK = 128
NBLK = N_OUT // BLK
BLK_PER_W = -(-NBLK // NW)
NU = BLK_PER_W // 2
NSLOT = 3

_mesh = plsc.VectorSubcoreMesh(core_axis_name="c", subcore_axis_name="s")


@functools.partial(
    pl.kernel,
    mesh=_mesh,
    out_type=jax.ShapeDtypeStruct((N_OUT, D), jnp.float32),
    scratch_types=[
        pltpu.VMEM((BLK_PER_W * BLK,), jnp.int32),
        pltpu.VMEM((NSLOT, 2 * BLK, D), jnp.float32),
        pltpu.SemaphoreType.DMA((NSLOT,)),
        pltpu.SemaphoreType.DMA((NSLOT,)),
    ],
)
def _gather_kernel(table_hbm, idx_hbm, out_hbm, idx_v, rows_v, gsem, wsem):
    wid = lax.axis_index("s") * NC + lax.axis_index("c")
    blk_start = jnp.minimum(wid * BLK_PER_W, NBLK - BLK_PER_W)
    pltpu.sync_copy(idx_hbm.at[pl.ds(blk_start * BLK, BLK_PER_W * BLK)], idx_v)

    def gathers_start(u, slot):
        for h in range(2):
            pltpu.async_copy(
                table_hbm.at[idx_v.at[pl.ds((2 * u + h) * BLK, BLK)]],
                rows_v.at[slot].at[pl.ds(h * BLK, BLK)],
                gsem.at[slot])

    def gathers_wait(u, slot):
        for h in range(2):
            pltpu.make_async_copy(
                table_hbm.at[idx_v.at[pl.ds(h * BLK, BLK)]],
                rows_v.at[slot].at[pl.ds(h * BLK, BLK)],
                gsem.at[slot]).wait()

    def write_start(u, slot):
        pltpu.async_copy(
            rows_v.at[slot],
            out_hbm.at[pl.ds((blk_start + 2 * u) * BLK, 2 * BLK)],
            wsem.at[slot])

    def write_wait(slot):
        pltpu.make_async_copy(
            rows_v.at[slot], out_hbm.at[pl.ds(0, 2 * BLK)],
            wsem.at[slot]).wait()

    gathers_start(0, 0)
    gathers_start(1, 1)
    gathers_wait(0, 0)
    write_start(0, 0)
    gathers_start(2, 2)

    def body(u, carry):
        b = lax.rem(u, NSLOT)
        gathers_wait(u, b)
        write_start(u, b)
        bn = lax.rem(u + 2, NSLOT)
        write_wait(bn)
        gathers_start(u + 2, bn)
        return carry

    lax.fori_loop(1, NU - 2, body, 0)

    for u in (NU - 2, NU - 1):
        b = u % NSLOT
        gathers_wait(u, b)
        write_start(u, b)
    for u in (NU - 3, NU - 2, NU - 1):
        write_wait(u % NSLOT)


def kernel(inputs, vt_replace, vt_map):
    del vt_replace
    return _gather_kernel(inputs, vt_map)


# speedup vs baseline: 1.0132x; 1.0132x over previous
import functools

import jax
import jax.numpy as jnp
from jax import lax
from jax.experimental import pallas as pl
from jax.experimental.pallas import tpu as pltpu
from jax.experimental.pallas import tpu_sc as plsc

N_OUT = 400000
D = 128
NC = 2
NS = 16
NW = NC * NS
BLK = 128
NBLK = N_OUT // BLK
BLK_PER_W = -(-NBLK // NW)

_mesh = plsc.VectorSubcoreMesh(core_axis_name="c", subcore_axis_name="s")

NBUF = 7
K = NBUF - 2


@functools.partial(
    pl.kernel,
    mesh=_mesh,
    out_type=jax.ShapeDtypeStruct((N_OUT, D), jnp.float32),
    scratch_types=[
        pltpu.VMEM((BLK_PER_W * BLK,), jnp.int32),
        pltpu.VMEM((NBUF, BLK, D), jnp.float32),
        pltpu.SemaphoreType.DMA((NBUF,)),
        pltpu.SemaphoreType.DMA((NBUF,)),
    ],
)
def _gather_kernel(table_hbm, idx_hbm, out_hbm, idx_v, rows_v, gsem, wsem):
    wid = lax.axis_index("s") * NC + lax.axis_index("c")
    blk0 = jnp.minimum(wid * BLK_PER_W, NBLK - BLK_PER_W)
    pltpu.sync_copy(idx_hbm.at[pl.ds(blk0 * BLK, BLK_PER_W * BLK)], idx_v)

    def gather_start(j, slot):
        pltpu.async_copy(
            table_hbm.at[idx_v.at[pl.ds(j * BLK, BLK)]],
            rows_v.at[slot], gsem.at[slot])

    def gather_wait(slot):
        pltpu.make_async_copy(
            table_hbm.at[idx_v.at[pl.ds(0, BLK)]],
            rows_v.at[slot], gsem.at[slot]).wait()

    def write_start(j, slot):
        pltpu.async_copy(
            rows_v.at[slot], out_hbm.at[pl.ds((blk0 + j) * BLK, BLK)],
            wsem.at[slot])

    def write_wait(slot):
        pltpu.make_async_copy(
            rows_v.at[slot], out_hbm.at[pl.ds(0, BLK)], wsem.at[slot]).wait()

    for jj in range(K):
        gather_start(jj, jj)

    def body(j, carry):
        b = lax.rem(j, NBUF)
        gather_wait(b)
        write_start(j, b)
        bn = lax.rem(j + K, NBUF)

        @pl.when(j + K - NBUF >= 0)
        def _():
            write_wait(bn)

        @pl.when(j + K < BLK_PER_W)
        def _():
            gather_start(j + K, bn)

        return carry

    lax.fori_loop(0, BLK_PER_W, body, 0)

    for u in range(BLK_PER_W - (NBUF - K), BLK_PER_W):
        write_wait(u % NBUF)


def kernel(inputs, vt_replace, vt_map):
    del vt_replace
    return _gather_kernel(inputs, vt_map)
